# baseline (speedup 1.0000x reference)
import functools

import jax
import jax.numpy as jnp
from jax import lax
from jax.experimental import pallas as pl
from jax.experimental.pallas import tpu as pltpu
from jax.experimental.pallas import tpu_sc as plsc

N = 10000
E = 320000
D = 128
G = 16

NW = 32
CHUNK = 128
NCH = 80
EP = NW * NCH * CHUNK
P = 10240
RPT = P // 16

_f32 = jnp.float32
_i32 = jnp.int32


def _mesh():
    return plsc.VectorSubcoreMesh(core_axis_name="c", subcore_axis_name="s")


def _deg_call(dstp, ones_rows, z16):
    @functools.partial(
        pl.kernel,
        out_type=jax.ShapeDtypeStruct((2, P, 16), _f32),
        mesh=_mesh(),
        compiler_params=pltpu.CompilerParams(use_tc_tiling_on_sc=False),
        scratch_types=[
            pltpu.VMEM((NCH, CHUNK), _i32),
            pltpu.VMEM((CHUNK, 16), _f32),
            pltpu.VMEM_SHARED((P, 16), _f32),
            pltpu.SemaphoreType.DMA,
        ],
    )
    def deg_kernel(dst_hbm, ones_hbm, z_hbm, out_hbm, dstv, onesv, acc, sd):
        c = lax.axis_index("c")
        s = lax.axis_index("s")
        wid = s * 2 + c
        pltpu.sync_copy(z_hbm, acc.at[pl.ds(s * RPT, RPT)])
        pltpu.sync_copy(ones_hbm, onesv)
        pltpu.sync_copy(dst_hbm.at[wid], dstv)
        plsc.subcore_barrier()

        def ebody(j, carry):
            pltpu.async_copy(onesv, acc.at[dstv.at[j]], sd, add=True)
            return carry

        lax.fori_loop(0, NCH, ebody, 0)

        def dbody(j, carry):
            pltpu.make_async_copy(onesv, acc.at[dstv.at[0]], sd).wait()
            return carry

        lax.fori_loop(0, NCH, dbody, 0)
        plsc.subcore_barrier()
        pltpu.sync_copy(acc.at[pl.ds(s * RPT, RPT)],
                        out_hbm.at[c, pl.ds(s * RPT, RPT)])

    return deg_kernel(dstp, ones_rows, z16)


def _msg_call(srcp, dstp, hp, zrows, F, stage):
    @functools.partial(
        pl.kernel,
        out_type=jax.ShapeDtypeStruct((2, P, F), _f32),
        mesh=_mesh(),
        compiler_params=pltpu.CompilerParams(use_tc_tiling_on_sc=False),
        scratch_types=[
            pltpu.VMEM((NCH, CHUNK), _i32),
            pltpu.VMEM((NCH, CHUNK), _i32),
            pltpu.VMEM((CHUNK, F), _f32),
            pltpu.VMEM((CHUNK, F), _f32),
            pltpu.VMEM((CHUNK, F), _f32),
            pltpu.VMEM((CHUNK, F), _f32),
            pltpu.VMEM_SHARED((P, F if stage else 1), _f32),
            pltpu.VMEM_SHARED((P, F), _f32),
            pltpu.SemaphoreType.DMA,
            pltpu.SemaphoreType.DMA,
            pltpu.SemaphoreType.DMA,
            pltpu.SemaphoreType.DMA,
            pltpu.SemaphoreType.DMA,
            pltpu.SemaphoreType.DMA,
            pltpu.SemaphoreType.DMA,
            pltpu.SemaphoreType.DMA,
        ],
    )
    def msg_kernel(src_hbm, dst_hbm, h_hbm, z_hbm, out_hbm,
                   srcv, dstv, g0, g1, g2, g3, hs, acc,
                   sg0, sg1, sg2, sg3, ss0, ss1, ss2, ss3):
        c = lax.axis_index("c")
        s = lax.axis_index("s")
        wid = s * 2 + c
        gb = [g0, g1, g2, g3]
        sg = [sg0, sg1, sg2, sg3]
        ss = [ss0, ss1, ss2, ss3]
        pltpu.sync_copy(z_hbm, acc.at[pl.ds(s * RPT, RPT)])
        if stage:
            pltpu.sync_copy(h_hbm.at[pl.ds(s * RPT, RPT)],
                            hs.at[pl.ds(s * RPT, RPT)])
        htab = hs if stage else h_hbm
        pltpu.sync_copy(src_hbm.at[wid], srcv)
        pltpu.sync_copy(dst_hbm.at[wid], dstv)
        plsc.subcore_barrier()

        for b in range(4):
            pltpu.async_copy(htab.at[srcv.at[b]], gb[b], sg[b])

        def body(g, carry):
            for b in range(4):
                j = g * 4 + b
                pltpu.make_async_copy(htab.at[srcv.at[j]], gb[b], sg[b]).wait()
                pltpu.async_copy(gb[b], acc.at[dstv.at[j]], ss[b], add=True)
                bp = (b + 3) & 3
                ok = jnp.logical_and(j >= 1, j <= NCH - 4)

                @pl.when(ok)
                def _():
                    pltpu.make_async_copy(gb[bp], acc.at[dstv.at[0]],
                                          ss[bp]).wait()
                    pltpu.async_copy(htab.at[srcv.at[j + 3]], gb[bp], sg[bp])

            return carry

        lax.fori_loop(0, NCH // 4, body, 0)
        for b in range(4):
            pltpu.make_async_copy(gb[b], acc.at[dstv.at[0]], ss[b]).wait()
        plsc.subcore_barrier()
        pltpu.sync_copy(acc.at[pl.ds(s * RPT, RPT)],
                        out_hbm.at[c, pl.ds(s * RPT, RPT)])

    return msg_kernel(srcp, dstp, hp, zrows)


def _msg2_call(srcp, dstp, h2a, h2b, zrows):
    F = 32

    @functools.partial(
        pl.kernel,
        out_type=jax.ShapeDtypeStruct((2, 2, P, F), _f32),
        mesh=_mesh(),
        compiler_params=pltpu.CompilerParams(use_tc_tiling_on_sc=False),
        scratch_types=[
            pltpu.VMEM((NCH, CHUNK), _i32),
            pltpu.VMEM((NCH, CHUNK), _i32),
            pltpu.VMEM((CHUNK, F), _f32),
            pltpu.VMEM((CHUNK, F), _f32),
            pltpu.VMEM((CHUNK, F), _f32),
            pltpu.VMEM((CHUNK, F), _f32),
            pltpu.VMEM_SHARED((P, F), _f32),
            pltpu.VMEM_SHARED((P, F), _f32),
            pltpu.SemaphoreType.DMA,
            pltpu.SemaphoreType.DMA,
            pltpu.SemaphoreType.DMA,
            pltpu.SemaphoreType.DMA,
            pltpu.SemaphoreType.DMA,
            pltpu.SemaphoreType.DMA,
            pltpu.SemaphoreType.DMA,
            pltpu.SemaphoreType.DMA,
        ],
    )
    def msg2_kernel(src_hbm, dst_hbm, ha_hbm, hb_hbm, z_hbm, out_hbm,
                    srcv, dstv, g0, g1, g2, g3, hs, acc,
                    sg0, sg1, sg2, sg3, ss0, ss1, ss2, ss3):
        c = lax.axis_index("c")
        s = lax.axis_index("s")
        wid = s * 2 + c
        gb = [g0, g1, g2, g3]
        sg = [sg0, sg1, sg2, sg3]
        ss = [ss0, ss1, ss2, ss3]
        halves = [ha_hbm, hb_hbm]
        pltpu.sync_copy(src_hbm.at[wid], srcv)
        pltpu.sync_copy(dst_hbm.at[wid], dstv)

        for half in range(2):
            pltpu.sync_copy(z_hbm, acc.at[pl.ds(s * RPT, RPT)])
            pltpu.sync_copy(halves[half].at[pl.ds(s * RPT, RPT)],
                            hs.at[pl.ds(s * RPT, RPT)])
            plsc.subcore_barrier()

            for b in range(4):
                pltpu.async_copy(hs.at[srcv.at[b]], gb[b], sg[b])

            def body(g, carry):
                for b in range(4):
                    j = g * 4 + b
                    pltpu.make_async_copy(hs.at[srcv.at[j]], gb[b],
                                          sg[b]).wait()
                    pltpu.async_copy(gb[b], acc.at[dstv.at[j]], ss[b],
                                     add=True)
                    bp = (b + 3) & 3
                    ok = jnp.logical_and(j >= 1, j <= NCH - 4)

                    @pl.when(ok)
                    def _():
                        pltpu.make_async_copy(gb[bp], acc.at[dstv.at[0]],
                                              ss[bp]).wait()
                        pltpu.async_copy(hs.at[srcv.at[j + 3]], gb[bp],
                                         sg[bp])

                return carry

            lax.fori_loop(0, NCH // 4, body, 0)
            for b in range(4):
                pltpu.make_async_copy(gb[b], acc.at[dstv.at[0]], ss[b]).wait()
            plsc.subcore_barrier()
            pltpu.sync_copy(acc.at[pl.ds(s * RPT, RPT)],
                            out_hbm.at[c, half, pl.ds(s * RPT, RPT)])

    return msg2_kernel(srcp, dstp, h2a, h2b, zrows)


def _tc_mm1(x_p, W1):
    def body(x_ref, w_ref, h_ref):
        h_ref[...] = jnp.dot(x_ref[...], w_ref[...],
                             preferred_element_type=_f32)

    return pl.pallas_call(
        body,
        out_shape=jax.ShapeDtypeStruct((P, W1.shape[1]), _f32),
    )(x_p, W1)


def _tc_scale(deg2, h1):
    def body(deg_ref, h1_ref, dinv_ref, h_ref):
        d = deg_ref[...]
        deg = (d[0] + d[1])[:, :1]
        dinv = lax.rsqrt(deg + 1.0)
        dinv_ref[...] = dinv
        h_ref[...] = h1_ref[...] * dinv

    return pl.pallas_call(
        body,
        out_shape=[jax.ShapeDtypeStruct((P, 1), _f32),
                   jax.ShapeDtypeStruct((P, h1.shape[1]), _f32)],
    )(deg2, h1)


def _tcB(s_parts, hp, dinv, b, g, be, Wn):
    Fn = Wn.shape[1]
    npart = len(s_parts)

    def body(*refs):
        s_refs = refs[:npart]
        (hp_ref, dinv_ref, b_ref, g_ref, be_ref, w_ref, out_ref) = refs[npart:]
        dinv = dinv_ref[...]
        stot = jnp.concatenate([r[0] + r[1] for r in s_refs], axis=1)
        pre = (stot + hp_ref[...]) * dinv + b_ref[...]
        rows = lax.broadcasted_iota(_i32, (P, 1), 0)
        m = rows < N
        prem = jnp.where(m, pre, 0.0)
        mean = jnp.sum(prem, axis=0, keepdims=True) * (1.0 / N)
        d = pre - mean
        var = jnp.sum(jnp.where(m, d * d, 0.0), axis=0, keepdims=True) * (1.0 / N)
        h = jnp.maximum(g_ref[...] * d * lax.rsqrt(var + 1e-5) + be_ref[...], 0.0)
        out_ref[...] = jnp.dot(h, w_ref[...], preferred_element_type=_f32) * dinv

    return pl.pallas_call(
        body,
        out_shape=jax.ShapeDtypeStruct((P, Fn), _f32),
    )(*s_parts, hp, dinv, b, g, be, Wn)


def _tcC(s2, hp, dinv, b, g, be, batch_col, Wf1, bf1, Wf2, bf2):
    def body(s_ref, hp_ref, dinv_ref, b_ref, g_ref, be_ref, batch_ref,
             wf1_ref, bf1_ref, wf2_ref, bf2_ref, out_ref):
        dinv = dinv_ref[...]
        pre = (s_ref[0] + s_ref[1] + hp_ref[...]) * dinv + b_ref[...]
        rows = lax.broadcasted_iota(_i32, (P, 1), 0)
        m = rows < N
        prem = jnp.where(m, pre, 0.0)
        mean = jnp.sum(prem, axis=0, keepdims=True) * (1.0 / N)
        d = pre - mean
        var = jnp.sum(jnp.where(m, d * d, 0.0), axis=0, keepdims=True) * (1.0 / N)
        h = jnp.maximum(g_ref[...] * d * lax.rsqrt(var + 1e-5) + be_ref[...], 0.0)
        bcol = batch_ref[...]
        sums = []
        cnts = []
        for gi in range(G):
            sel = bcol == gi
            sums.append(jnp.sum(jnp.where(sel, h, 0.0), axis=0, keepdims=True))
            cnts.append(jnp.sum(jnp.where(sel, 1.0, 0.0), axis=0, keepdims=True))
        pooled = jnp.concatenate(sums, axis=0) / jnp.maximum(
            jnp.concatenate(cnts, axis=0), 1.0)
        o = jnp.maximum(jnp.dot(pooled, wf1_ref[...],
                                preferred_element_type=_f32) + bf1_ref[...], 0.0)
        out_ref[...] = jnp.dot(o, wf2_ref[...],
                               preferred_element_type=_f32) + bf2_ref[...]

    return pl.pallas_call(
        body,
        out_shape=jax.ShapeDtypeStruct((G, 1), _f32),
    )(s2, hp, dinv, b, g, be, batch_col, Wf1, bf1, Wf2, bf2)


def kernel(x, edge_index, batch, W1, b1, g1, be1, W2, b2, g2, be2,
           W3, b3, g3, be3, Wf1, bf1, Wf2, bf2):
    src = edge_index[0]
    dst = edge_index[1]
    pad = jnp.full((EP - E,), N, _i32)
    srcp = jnp.concatenate([src, pad]).reshape(NW, NCH, CHUNK)
    dstp = jnp.concatenate([dst, pad]).reshape(NW, NCH, CHUNK)
    x_p = jnp.zeros((P, D), _f32).at[:N].set(x)
    z16 = jnp.zeros((RPT, 16), _f32)
    z32 = jnp.zeros((RPT, 32), _f32)
    z64 = jnp.zeros((RPT, 64), _f32)
    ones_rows = jnp.ones((CHUNK, 16), _f32)

    deg2 = _deg_call(dstp, ones_rows, z16)
    h1 = _tc_mm1(x_p, W1)

    dinv, h1p = _tc_scale(deg2, h1)
    s1 = _msg_call(srcp, dstp, h1p, z32, 32, True)
    h2p = _tcB([s1], h1p, dinv, b1.reshape(1, -1), g1.reshape(1, -1),
               be1.reshape(1, -1), W2)
    s2ab = _msg2_call(srcp, dstp, h2p[:, :32], h2p[:, 32:], z32)
    h3p = _tcB([s2ab[:, 0], s2ab[:, 1]], h2p, dinv,
               b2.reshape(1, -1), g2.reshape(1, -1),
               be2.reshape(1, -1), W3)
    s3 = _msg_call(srcp, dstp, h3p, z32, 32, True)

    batch_col = jnp.full((P,), -1, _i32).at[:N].set(batch).reshape(P, 1)
    return _tcC(s3, h3p, dinv, b3.reshape(1, -1), g3.reshape(1, -1),
                be3.reshape(1, -1), batch_col, Wf1, bf1.reshape(1, -1),
                Wf2, bf2.reshape(1, 1))


# speedup vs baseline: 1.0106x; 1.0106x over previous
import functools

import jax
import jax.numpy as jnp
from jax import lax
from jax.experimental import pallas as pl
from jax.experimental.pallas import tpu as pltpu
from jax.experimental.pallas import tpu_sc as plsc

N = 10000
E = 320000
D = 128
G = 16

NW = 32
CHUNK = 128
NCH = 80
EP = NW * NCH * CHUNK
P = 10240
RPT = P // 16

_f32 = jnp.float32
_i32 = jnp.int32


def _mesh():
    return plsc.VectorSubcoreMesh(core_axis_name="c", subcore_axis_name="s")


def _deg_call(dstp, ones_rows, z16):
    @functools.partial(
        pl.kernel,
        out_type=jax.ShapeDtypeStruct((2, P, 16), _f32),
        mesh=_mesh(),
        compiler_params=pltpu.CompilerParams(use_tc_tiling_on_sc=False),
        scratch_types=[
            pltpu.VMEM((NCH, CHUNK), _i32),
            pltpu.VMEM((CHUNK, 16), _f32),
            pltpu.VMEM_SHARED((P, 16), _f32),
            pltpu.SemaphoreType.DMA,
        ],
    )
    def deg_kernel(dst_hbm, ones_hbm, z_hbm, out_hbm, dstv, onesv, acc, sd):
        c = lax.axis_index("c")
        s = lax.axis_index("s")
        wid = s * 2 + c
        pltpu.sync_copy(z_hbm, acc.at[pl.ds(s * RPT, RPT)])
        pltpu.sync_copy(ones_hbm, onesv)
        pltpu.sync_copy(dst_hbm.at[wid], dstv)
        plsc.subcore_barrier()

        def ebody(j, carry):
            pltpu.async_copy(onesv, acc.at[dstv.at[j]], sd, add=True)
            return carry

        lax.fori_loop(0, NCH, ebody, 0)

        def dbody(j, carry):
            pltpu.make_async_copy(onesv, acc.at[dstv.at[0]], sd).wait()
            return carry

        lax.fori_loop(0, NCH, dbody, 0)
        plsc.subcore_barrier()
        pltpu.sync_copy(acc.at[pl.ds(s * RPT, RPT)],
                        out_hbm.at[c, pl.ds(s * RPT, RPT)])

    return deg_kernel(dstp, ones_rows, z16)


def _msg_call(srcp, dstp, hp, zrows, F, stage):
    @functools.partial(
        pl.kernel,
        out_type=jax.ShapeDtypeStruct((2, P, F), _f32),
        mesh=_mesh(),
        compiler_params=pltpu.CompilerParams(use_tc_tiling_on_sc=False),
        scratch_types=[
            pltpu.VMEM((NCH, CHUNK), _i32),
            pltpu.VMEM((NCH, CHUNK), _i32),
            pltpu.VMEM((CHUNK, F), _f32),
            pltpu.VMEM((CHUNK, F), _f32),
            pltpu.VMEM((CHUNK, F), _f32),
            pltpu.VMEM((CHUNK, F), _f32),
            pltpu.VMEM_SHARED((P, F if stage else 1), _f32),
            pltpu.VMEM_SHARED((P, F), _f32),
            pltpu.SemaphoreType.DMA,
            pltpu.SemaphoreType.DMA,
            pltpu.SemaphoreType.DMA,
            pltpu.SemaphoreType.DMA,
            pltpu.SemaphoreType.DMA,
            pltpu.SemaphoreType.DMA,
            pltpu.SemaphoreType.DMA,
            pltpu.SemaphoreType.DMA,
        ],
    )
    def msg_kernel(src_hbm, dst_hbm, h_hbm, z_hbm, out_hbm,
                   srcv, dstv, g0, g1, g2, g3, hs, acc,
                   sg0, sg1, sg2, sg3, ss0, ss1, ss2, ss3):
        c = lax.axis_index("c")
        s = lax.axis_index("s")
        wid = s * 2 + c
        gb = [g0, g1, g2, g3]
        sg = [sg0, sg1, sg2, sg3]
        ss = [ss0, ss1, ss2, ss3]
        pltpu.sync_copy(z_hbm, acc.at[pl.ds(s * RPT, RPT)])
        if stage:
            pltpu.sync_copy(h_hbm.at[pl.ds(s * RPT, RPT)],
                            hs.at[pl.ds(s * RPT, RPT)])
        htab = hs if stage else h_hbm
        pltpu.sync_copy(src_hbm.at[wid], srcv)
        pltpu.sync_copy(dst_hbm.at[wid], dstv)
        plsc.subcore_barrier()

        for b in range(4):
            pltpu.async_copy(htab.at[srcv.at[b]], gb[b], sg[b])

        def body(g, carry):
            for b in range(4):
                j = g * 4 + b
                pltpu.make_async_copy(htab.at[srcv.at[j]], gb[b], sg[b]).wait()
                pltpu.async_copy(gb[b], acc.at[dstv.at[j]], ss[b], add=True)
                bp = (b + 3) & 3
                ok = jnp.logical_and(j >= 1, j <= NCH - 4)

                @pl.when(ok)
                def _():
                    pltpu.make_async_copy(gb[bp], acc.at[dstv.at[0]],
                                          ss[bp]).wait()
                    pltpu.async_copy(htab.at[srcv.at[j + 3]], gb[bp], sg[bp])

            return carry

        lax.fori_loop(0, NCH // 4, body, 0)
        for b in range(4):
            pltpu.make_async_copy(gb[b], acc.at[dstv.at[0]], ss[b]).wait()
        plsc.subcore_barrier()
        pltpu.sync_copy(acc.at[pl.ds(s * RPT, RPT)],
                        out_hbm.at[c, pl.ds(s * RPT, RPT)])

    return msg_kernel(srcp, dstp, hp, zrows)


def _msg2_call(srcp, dstp, h2a, h2b, zrows):
    F = 32

    @functools.partial(
        pl.kernel,
        out_type=jax.ShapeDtypeStruct((2, 2, P, F), _f32),
        mesh=_mesh(),
        compiler_params=pltpu.CompilerParams(use_tc_tiling_on_sc=False),
        scratch_types=[
            pltpu.VMEM((NCH, CHUNK), _i32),
            pltpu.VMEM((NCH, CHUNK), _i32),
            pltpu.VMEM((CHUNK, F), _f32),
            pltpu.VMEM((CHUNK, F), _f32),
            pltpu.VMEM((CHUNK, F), _f32),
            pltpu.VMEM((CHUNK, F), _f32),
            pltpu.VMEM_SHARED((P, F), _f32),
            pltpu.VMEM_SHARED((P, F), _f32),
            pltpu.SemaphoreType.DMA,
            pltpu.SemaphoreType.DMA,
            pltpu.SemaphoreType.DMA,
            pltpu.SemaphoreType.DMA,
            pltpu.SemaphoreType.DMA,
            pltpu.SemaphoreType.DMA,
            pltpu.SemaphoreType.DMA,
            pltpu.SemaphoreType.DMA,
        ],
    )
    def msg2_kernel(src_hbm, dst_hbm, ha_hbm, hb_hbm, z_hbm, out_hbm,
                    srcv, dstv, g0, g1, g2, g3, hs, acc,
                    sg0, sg1, sg2, sg3, ss0, ss1, ss2, ss3):
        c = lax.axis_index("c")
        s = lax.axis_index("s")
        wid = s * 2 + c
        gb = [g0, g1, g2, g3]
        sg = [sg0, sg1, sg2, sg3]
        ss = [ss0, ss1, ss2, ss3]
        halves = [ha_hbm, hb_hbm]
        pltpu.sync_copy(src_hbm.at[wid], srcv)
        pltpu.sync_copy(dst_hbm.at[wid], dstv)

        for half in range(2):
            pltpu.sync_copy(z_hbm, acc.at[pl.ds(s * RPT, RPT)])
            pltpu.sync_copy(halves[half].at[pl.ds(s * RPT, RPT)],
                            hs.at[pl.ds(s * RPT, RPT)])
            plsc.subcore_barrier()

            for b in range(4):
                pltpu.async_copy(hs.at[srcv.at[b]], gb[b], sg[b])

            def body(g, carry):
                for b in range(4):
                    j = g * 4 + b
                    pltpu.make_async_copy(hs.at[srcv.at[j]], gb[b],
                                          sg[b]).wait()
                    pltpu.async_copy(gb[b], acc.at[dstv.at[j]], ss[b],
                                     add=True)
                    bp = (b + 3) & 3
                    ok = jnp.logical_and(j >= 1, j <= NCH - 4)

                    @pl.when(ok)
                    def _():
                        pltpu.make_async_copy(gb[bp], acc.at[dstv.at[0]],
                                              ss[bp]).wait()
                        pltpu.async_copy(hs.at[srcv.at[j + 3]], gb[bp],
                                         sg[bp])

                return carry

            lax.fori_loop(0, NCH // 4, body, 0)
            for b in range(4):
                pltpu.make_async_copy(gb[b], acc.at[dstv.at[0]], ss[b]).wait()
            plsc.subcore_barrier()
            pltpu.sync_copy(acc.at[pl.ds(s * RPT, RPT)],
                            out_hbm.at[c, half, pl.ds(s * RPT, RPT)])

    return msg2_kernel(srcp, dstp, h2a, h2b, zrows)


def _tcA(deg2, x_p, W1):
    def body(deg_ref, x_ref, w_ref, dinv_ref, h_ref):
        d = deg_ref[...]
        deg = (d[0] + d[1])[:, :1]
        dinv = lax.rsqrt(deg + 1.0)
        dinv_ref[...] = dinv
        h_ref[...] = jnp.dot(x_ref[...], w_ref[...],
                             preferred_element_type=_f32) * dinv

    return pl.pallas_call(
        body,
        out_shape=[jax.ShapeDtypeStruct((P, 1), _f32),
                   jax.ShapeDtypeStruct((P, W1.shape[1]), _f32)],
    )(deg2, x_p, W1)


def _tcB(s_parts, hp, dinv, b, g, be, Wn):
    Fn = Wn.shape[1]
    npart = len(s_parts)

    def body(*refs):
        s_refs = refs[:npart]
        (hp_ref, dinv_ref, b_ref, g_ref, be_ref, w_ref, out_ref) = refs[npart:]
        dinv = dinv_ref[...]
        stot = jnp.concatenate([r[0] + r[1] for r in s_refs], axis=1)
        pre = (stot + hp_ref[...]) * dinv + b_ref[...]
        rows = lax.broadcasted_iota(_i32, (P, 1), 0)
        m = rows < N
        prem = jnp.where(m, pre, 0.0)
        mean = jnp.sum(prem, axis=0, keepdims=True) * (1.0 / N)
        d = pre - mean
        var = jnp.sum(jnp.where(m, d * d, 0.0), axis=0, keepdims=True) * (1.0 / N)
        h = jnp.maximum(g_ref[...] * d * lax.rsqrt(var + 1e-5) + be_ref[...], 0.0)
        out_ref[...] = jnp.dot(h, w_ref[...], preferred_element_type=_f32) * dinv

    return pl.pallas_call(
        body,
        out_shape=jax.ShapeDtypeStruct((P, Fn), _f32),
    )(*s_parts, hp, dinv, b, g, be, Wn)


def _tcC(s2, hp, dinv, b, g, be, batch_col, Wf1, bf1, Wf2, bf2):
    def body(s_ref, hp_ref, dinv_ref, b_ref, g_ref, be_ref, batch_ref,
             wf1_ref, bf1_ref, wf2_ref, bf2_ref, out_ref):
        dinv = dinv_ref[...]
        pre = (s_ref[0] + s_ref[1] + hp_ref[...]) * dinv + b_ref[...]
        rows = lax.broadcasted_iota(_i32, (P, 1), 0)
        m = rows < N
        prem = jnp.where(m, pre, 0.0)
        mean = jnp.sum(prem, axis=0, keepdims=True) * (1.0 / N)
        d = pre - mean
        var = jnp.sum(jnp.where(m, d * d, 0.0), axis=0, keepdims=True) * (1.0 / N)
        h = jnp.maximum(g_ref[...] * d * lax.rsqrt(var + 1e-5) + be_ref[...], 0.0)
        bcol = batch_ref[...]
        sums = []
        cnts = []
        for gi in range(G):
            sel = bcol == gi
            sums.append(jnp.sum(jnp.where(sel, h, 0.0), axis=0, keepdims=True))
            cnts.append(jnp.sum(jnp.where(sel, 1.0, 0.0), axis=0, keepdims=True))
        pooled = jnp.concatenate(sums, axis=0) / jnp.maximum(
            jnp.concatenate(cnts, axis=0), 1.0)
        o = jnp.maximum(jnp.dot(pooled, wf1_ref[...],
                                preferred_element_type=_f32) + bf1_ref[...], 0.0)
        out_ref[...] = jnp.dot(o, wf2_ref[...],
                               preferred_element_type=_f32) + bf2_ref[...]

    return pl.pallas_call(
        body,
        out_shape=jax.ShapeDtypeStruct((G, 1), _f32),
    )(s2, hp, dinv, b, g, be, batch_col, Wf1, bf1, Wf2, bf2)


def kernel(x, edge_index, batch, W1, b1, g1, be1, W2, b2, g2, be2,
           W3, b3, g3, be3, Wf1, bf1, Wf2, bf2):
    src = edge_index[0]
    dst = edge_index[1]
    pad = jnp.full((EP - E,), N, _i32)
    srcp = jnp.concatenate([src, pad]).reshape(NW, NCH, CHUNK)
    dstp = jnp.concatenate([dst, pad]).reshape(NW, NCH, CHUNK)
    x_p = jnp.zeros((P, D), _f32).at[:N].set(x)
    z16 = jnp.zeros((RPT, 16), _f32)
    z32 = jnp.zeros((RPT, 32), _f32)
    z64 = jnp.zeros((RPT, 64), _f32)
    ones_rows = jnp.ones((CHUNK, 16), _f32)

    deg2 = _deg_call(dstp, ones_rows, z16)

    dinv, h1p = _tcA(deg2, x_p, W1)
    s1 = _msg_call(srcp, dstp, h1p, z32, 32, True)
    h2p = _tcB([s1], h1p, dinv, b1.reshape(1, -1), g1.reshape(1, -1),
               be1.reshape(1, -1), W2)
    s2ab = _msg2_call(srcp, dstp, h2p[:, :32], h2p[:, 32:], z32)
    h3p = _tcB([s2ab[:, 0], s2ab[:, 1]], h2p, dinv,
               b2.reshape(1, -1), g2.reshape(1, -1),
               be2.reshape(1, -1), W3)
    s3 = _msg_call(srcp, dstp, h3p, z32, 32, True)

    batch_col = jnp.full((P,), -1, _i32).at[:N].set(batch).reshape(P, 1)
    return _tcC(s3, h3p, dinv, b3.reshape(1, -1), g3.reshape(1, -1),
                be3.reshape(1, -1), batch_col, Wf1, bf1.reshape(1, -1),
                Wf2, bf2.reshape(1, 1))


# speedup vs baseline: 1.0274x; 1.0166x over previous
import functools

import jax
import jax.numpy as jnp
from jax import lax
from jax.experimental import pallas as pl
from jax.experimental.pallas import tpu as pltpu
from jax.experimental.pallas import tpu_sc as plsc

N = 10000
E = 320000
D = 128
G = 16

NW = 32
CHUNK = 128
NCH = 80
EP = NW * NCH * CHUNK
P = 10240
RPT = P // 16

_f32 = jnp.float32
_i32 = jnp.int32


def _mesh():
    return plsc.VectorSubcoreMesh(core_axis_name="c", subcore_axis_name="s")


def _deg_call(dstp, ones_rows, z16):
    @functools.partial(
        pl.kernel,
        out_type=jax.ShapeDtypeStruct((2, P, 16), _f32),
        mesh=_mesh(),
        compiler_params=pltpu.CompilerParams(use_tc_tiling_on_sc=False),
        scratch_types=[
            pltpu.VMEM((NCH, CHUNK), _i32),
            pltpu.VMEM((CHUNK, 16), _f32),
            pltpu.VMEM_SHARED((P, 16), _f32),
            pltpu.SemaphoreType.DMA,
        ],
    )
    def deg_kernel(dst_hbm, ones_hbm, z_hbm, out_hbm, dstv, onesv, acc, sd):
        c = lax.axis_index("c")
        s = lax.axis_index("s")
        wid = s * 2 + c
        pltpu.sync_copy(z_hbm, acc.at[pl.ds(s * RPT, RPT)])
        pltpu.sync_copy(ones_hbm, onesv)
        pltpu.sync_copy(dst_hbm.at[wid], dstv)
        plsc.subcore_barrier()

        def ebody(j, carry):
            pltpu.async_copy(onesv, acc.at[dstv.at[j]], sd, add=True)
            return carry

        lax.fori_loop(0, NCH, ebody, 0)

        def dbody(j, carry):
            pltpu.make_async_copy(onesv, acc.at[dstv.at[0]], sd).wait()
            return carry

        lax.fori_loop(0, NCH, dbody, 0)
        plsc.subcore_barrier()
        pltpu.sync_copy(acc.at[pl.ds(s * RPT, RPT)],
                        out_hbm.at[c, pl.ds(s * RPT, RPT)])

    return deg_kernel(dstp, ones_rows, z16)


def _msg_call(srcp, dstp, hp, zrows, F, stage):
    @functools.partial(
        pl.kernel,
        out_type=jax.ShapeDtypeStruct((2, P, F), _f32),
        mesh=_mesh(),
        compiler_params=pltpu.CompilerParams(use_tc_tiling_on_sc=False),
        scratch_types=[
            pltpu.VMEM((NCH, CHUNK), _i32),
            pltpu.VMEM((NCH, CHUNK), _i32),
            pltpu.VMEM((CHUNK, F), _f32),
            pltpu.VMEM((CHUNK, F), _f32),
            pltpu.VMEM((CHUNK, F), _f32),
            pltpu.VMEM((CHUNK, F), _f32),
            pltpu.VMEM_SHARED((P, F if stage else 1), _f32),
            pltpu.VMEM_SHARED((P, F), _f32),
            pltpu.SemaphoreType.DMA,
            pltpu.SemaphoreType.DMA,
            pltpu.SemaphoreType.DMA,
            pltpu.SemaphoreType.DMA,
            pltpu.SemaphoreType.DMA,
            pltpu.SemaphoreType.DMA,
            pltpu.SemaphoreType.DMA,
            pltpu.SemaphoreType.DMA,
        ],
    )
    def msg_kernel(src_hbm, dst_hbm, h_hbm, z_hbm, out_hbm,
                   srcv, dstv, g0, g1, g2, g3, hs, acc,
                   sg0, sg1, sg2, sg3, ss0, ss1, ss2, ss3):
        c = lax.axis_index("c")
        s = lax.axis_index("s")
        wid = s * 2 + c
        gb = [g0, g1, g2, g3]
        sg = [sg0, sg1, sg2, sg3]
        ss = [ss0, ss1, ss2, ss3]
        pltpu.sync_copy(z_hbm, acc.at[pl.ds(s * RPT, RPT)])
        if stage:
            pltpu.sync_copy(h_hbm.at[pl.ds(s * RPT, RPT)],
                            hs.at[pl.ds(s * RPT, RPT)])
        htab = hs if stage else h_hbm
        pltpu.sync_copy(src_hbm.at[wid], srcv)
        pltpu.sync_copy(dst_hbm.at[wid], dstv)
        plsc.subcore_barrier()

        for b in range(4):
            pltpu.async_copy(htab.at[srcv.at[b]], gb[b], sg[b])

        def body(g, carry):
            for b in range(4):
                j = g * 4 + b
                pltpu.make_async_copy(htab.at[srcv.at[j]], gb[b], sg[b]).wait()
                pltpu.async_copy(gb[b], acc.at[dstv.at[j]], ss[b], add=True)
                bp = (b + 3) & 3
                ok = jnp.logical_and(j >= 1, j <= NCH - 4)

                @pl.when(ok)
                def _():
                    pltpu.make_async_copy(gb[bp], acc.at[dstv.at[0]],
                                          ss[bp]).wait()
                    pltpu.async_copy(htab.at[srcv.at[j + 3]], gb[bp], sg[bp])

            return carry

        lax.fori_loop(0, NCH // 4, body, 0)
        for b in range(4):
            pltpu.make_async_copy(gb[b], acc.at[dstv.at[0]], ss[b]).wait()
        plsc.subcore_barrier()
        pltpu.sync_copy(acc.at[pl.ds(s * RPT, RPT)],
                        out_hbm.at[c, pl.ds(s * RPT, RPT)])

    return msg_kernel(srcp, dstp, hp, zrows)


def _msg2_call(srcp, dstp, h2a, h2b, zrows):
    F = 32

    @functools.partial(
        pl.kernel,
        out_type=jax.ShapeDtypeStruct((2, 2, P, F), _f32),
        mesh=_mesh(),
        compiler_params=pltpu.CompilerParams(use_tc_tiling_on_sc=False),
        scratch_types=[
            pltpu.VMEM((NCH, CHUNK), _i32),
            pltpu.VMEM((NCH, CHUNK), _i32),
            pltpu.VMEM((CHUNK, F), _f32),
            pltpu.VMEM((CHUNK, F), _f32),
            pltpu.VMEM((CHUNK, F), _f32),
            pltpu.VMEM((CHUNK, F), _f32),
            pltpu.VMEM_SHARED((P, F), _f32),
            pltpu.VMEM_SHARED((P, F), _f32),
            pltpu.SemaphoreType.DMA,
            pltpu.SemaphoreType.DMA,
            pltpu.SemaphoreType.DMA,
            pltpu.SemaphoreType.DMA,
            pltpu.SemaphoreType.DMA,
            pltpu.SemaphoreType.DMA,
            pltpu.SemaphoreType.DMA,
            pltpu.SemaphoreType.DMA,
        ],
    )
    def msg2_kernel(src_hbm, dst_hbm, ha_hbm, hb_hbm, z_hbm, out_hbm,
                    srcv, dstv, g0, g1, g2, g3, hs, acc,
                    sg0, sg1, sg2, sg3, ss0, ss1, ss2, ss3):
        c = lax.axis_index("c")
        s = lax.axis_index("s")
        wid = s * 2 + c
        gb = [g0, g1, g2, g3]
        sg = [sg0, sg1, sg2, sg3]
        ss = [ss0, ss1, ss2, ss3]
        halves = [ha_hbm, hb_hbm]
        pltpu.sync_copy(src_hbm.at[wid], srcv)
        pltpu.sync_copy(dst_hbm.at[wid], dstv)

        for half in range(2):
            pltpu.sync_copy(z_hbm, acc.at[pl.ds(s * RPT, RPT)])
            pltpu.sync_copy(halves[half].at[pl.ds(s * RPT, RPT)],
                            hs.at[pl.ds(s * RPT, RPT)])
            plsc.subcore_barrier()

            for b in range(4):
                pltpu.async_copy(hs.at[srcv.at[b]], gb[b], sg[b])

            def body(g, carry):
                for b in range(4):
                    j = g * 4 + b
                    pltpu.make_async_copy(hs.at[srcv.at[j]], gb[b],
                                          sg[b]).wait()
                    pltpu.async_copy(gb[b], acc.at[dstv.at[j]], ss[b],
                                     add=True)
                    bp = (b + 3) & 3
                    ok = jnp.logical_and(j >= 1, j <= NCH - 4)

                    @pl.when(ok)
                    def _():
                        pltpu.make_async_copy(gb[bp], acc.at[dstv.at[0]],
                                              ss[bp]).wait()
                        pltpu.async_copy(hs.at[srcv.at[j + 3]], gb[bp],
                                         sg[bp])

                return carry

            lax.fori_loop(0, NCH // 4, body, 0)
            for b in range(4):
                pltpu.make_async_copy(gb[b], acc.at[dstv.at[0]], ss[b]).wait()
            plsc.subcore_barrier()
            pltpu.sync_copy(acc.at[pl.ds(s * RPT, RPT)],
                            out_hbm.at[c, half, pl.ds(s * RPT, RPT)])

    return msg2_kernel(srcp, dstp, h2a, h2b, zrows)


def _tcA(deg2, x_p, W1):
    def body(deg_ref, x_ref, w_ref, dinv_ref, h_ref):
        d = deg_ref[...]
        deg = (d[0] + d[1])[:, :1]
        dinv = lax.rsqrt(deg + 1.0)
        dinv_ref[...] = dinv
        h_ref[...] = jnp.dot(x_ref[...], w_ref[...],
                             preferred_element_type=_f32) * dinv

    return pl.pallas_call(
        body,
        out_shape=[jax.ShapeDtypeStruct((P, 1), _f32),
                   jax.ShapeDtypeStruct((P, W1.shape[1]), _f32)],
    )(deg2, x_p, W1)


def _tcB(s_parts, hp, dinv, b, g, be, Wn):
    Fn = Wn.shape[1]
    npart = len(s_parts)

    def body(*refs):
        s_refs = refs[:npart]
        (hp_ref, dinv_ref, b_ref, g_ref, be_ref, w_ref, out_ref) = refs[npart:]
        dinv = dinv_ref[...]
        stot = jnp.concatenate([r[0] + r[1] for r in s_refs], axis=1)
        pre = (stot + hp_ref[...]) * dinv + b_ref[...]
        rows = lax.broadcasted_iota(_i32, (P, 1), 0)
        m = rows < N
        prem = jnp.where(m, pre, 0.0)
        mean = jnp.sum(prem, axis=0, keepdims=True) * (1.0 / N)
        d = pre - mean
        var = jnp.sum(jnp.where(m, d * d, 0.0), axis=0, keepdims=True) * (1.0 / N)
        h = jnp.maximum(g_ref[...] * d * lax.rsqrt(var + 1e-5) + be_ref[...], 0.0)
        out_ref[...] = jnp.dot(h, w_ref[...], preferred_element_type=_f32) * dinv

    return pl.pallas_call(
        body,
        out_shape=jax.ShapeDtypeStruct((P, Fn), _f32),
    )(*s_parts, hp, dinv, b, g, be, Wn)


def _tcC(s2, hp, dinv, b, g, be, batch_col, Wf1, bf1, Wf2, bf2):
    def body(s_ref, hp_ref, dinv_ref, b_ref, g_ref, be_ref, batch_ref,
             wf1_ref, bf1_ref, wf2_ref, bf2_ref, out_ref):
        dinv = dinv_ref[...]
        pre = (s_ref[0] + s_ref[1] + hp_ref[...]) * dinv + b_ref[...]
        rows = lax.broadcasted_iota(_i32, (P, 1), 0)
        m = rows < N
        prem = jnp.where(m, pre, 0.0)
        mean = jnp.sum(prem, axis=0, keepdims=True) * (1.0 / N)
        d = pre - mean
        var = jnp.sum(jnp.where(m, d * d, 0.0), axis=0, keepdims=True) * (1.0 / N)
        h = jnp.maximum(g_ref[...] * d * lax.rsqrt(var + 1e-5) + be_ref[...], 0.0)
        bcol = batch_ref[...]
        sums = []
        cnts = []
        for gi in range(G):
            sel = bcol == gi
            sums.append(jnp.sum(jnp.where(sel, h, 0.0), axis=0, keepdims=True))
            cnts.append(jnp.sum(jnp.where(sel, 1.0, 0.0), axis=0, keepdims=True))
        pooled = jnp.concatenate(sums, axis=0) / jnp.maximum(
            jnp.concatenate(cnts, axis=0), 1.0)
        o = jnp.maximum(jnp.dot(pooled, wf1_ref[...],
                                preferred_element_type=_f32) + bf1_ref[...], 0.0)
        out_ref[...] = jnp.dot(o, wf2_ref[...],
                               preferred_element_type=_f32) + bf2_ref[...]

    return pl.pallas_call(
        body,
        out_shape=jax.ShapeDtypeStruct((G, 1), _f32),
    )(s2, hp, dinv, b, g, be, batch_col, Wf1, bf1, Wf2, bf2)


def kernel(x, edge_index, batch, W1, b1, g1, be1, W2, b2, g2, be2,
           W3, b3, g3, be3, Wf1, bf1, Wf2, bf2):
    src = edge_index[0]
    dst = edge_index[1]
    pad = jnp.full((EP - E,), N, _i32)
    srcp = jnp.concatenate([src, pad]).reshape(NW, NCH, CHUNK)
    dstp = jnp.concatenate([dst, pad]).reshape(NW, NCH, CHUNK)
    x_p = jnp.zeros((P, D), _f32).at[:N].set(x)
    z16 = jnp.zeros((RPT, 16), _f32)
    z32 = jnp.zeros((RPT, 32), _f32)
    z64 = jnp.zeros((RPT, 64), _f32)
    ones_rows = jnp.ones((CHUNK, 16), _f32)

    deg2 = _deg_call(dstp, ones_rows, z16)

    dinv, h1p = _tcA(deg2, x_p, W1)
    s1 = _msg_call(srcp, dstp, h1p, z32, 32, True)
    h2p = _tcB([s1], h1p, dinv, b1.reshape(1, -1), g1.reshape(1, -1),
               be1.reshape(1, -1), W2)
    s2a = _msg_call(srcp, dstp, h2p[:, :32], z32, 32, True)
    s2b = _msg_call(srcp, dstp, h2p[:, 32:], z32, 32, True)
    h3p = _tcB([s2a, s2b], h2p, dinv,
               b2.reshape(1, -1), g2.reshape(1, -1),
               be2.reshape(1, -1), W3)
    s3 = _msg_call(srcp, dstp, h3p, z32, 32, True)

    batch_col = jnp.full((P,), -1, _i32).at[:N].set(batch).reshape(P, 1)
    return _tcC(s3, h3p, dinv, b3.reshape(1, -1), g3.reshape(1, -1),
                be3.reshape(1, -1), batch_col, Wf1, bf1.reshape(1, -1),
                Wf2, bf2.reshape(1, 1))


# speedup vs baseline: 1.0658x; 1.0374x over previous
import functools

import jax
import jax.numpy as jnp
from jax import lax
from jax.experimental import pallas as pl
from jax.experimental.pallas import tpu as pltpu
from jax.experimental.pallas import tpu_sc as plsc

N = 10000
E = 320000
D = 128
G = 16

NW = 32
CHUNK = 128
NCH = 80
EP = NW * NCH * CHUNK
P = 10240
RPT = P // 16

_f32 = jnp.float32
_i32 = jnp.int32


def _mesh():
    return plsc.VectorSubcoreMesh(core_axis_name="c", subcore_axis_name="s")


def _deg_call(dstp, ones_rows, z16):
    @functools.partial(
        pl.kernel,
        out_type=jax.ShapeDtypeStruct((2, P, 8), _f32),
        mesh=_mesh(),
        compiler_params=pltpu.CompilerParams(use_tc_tiling_on_sc=False),
        scratch_types=[
            pltpu.VMEM((NCH, CHUNK), _i32),
            pltpu.VMEM((CHUNK, 8), _f32),
            pltpu.VMEM_SHARED((P, 8), _f32),
            pltpu.SemaphoreType.DMA,
            pltpu.SemaphoreType.DMA,
            pltpu.SemaphoreType.DMA,
        ],
    )
    def deg_kernel(dst_hbm, ones_hbm, z_hbm, out_hbm, dstv, onesv, acc,
                   sd, sp0, sp1):
        c = lax.axis_index("c")
        s = lax.axis_index("s")
        wid = s * 2 + c
        d0 = pltpu.async_copy(z_hbm, acc.at[pl.ds(s * RPT, RPT)], sp0)
        d1 = pltpu.async_copy(ones_hbm, onesv, sp1)
        d2 = pltpu.async_copy(dst_hbm.at[wid], dstv, sd)
        d0.wait()
        d1.wait()
        d2.wait()
        plsc.subcore_barrier()

        def ebody(j, carry):
            pltpu.async_copy(onesv, acc.at[dstv.at[j]], sd, add=True)
            return carry

        lax.fori_loop(0, NCH, ebody, 0)

        def dbody(j, carry):
            pltpu.make_async_copy(onesv, acc.at[dstv.at[0]], sd).wait()
            return carry

        lax.fori_loop(0, NCH, dbody, 0)
        plsc.subcore_barrier()
        pltpu.sync_copy(acc.at[pl.ds(s * RPT, RPT)],
                        out_hbm.at[c, pl.ds(s * RPT, RPT)])

    return deg_kernel(dstp, ones_rows, z16)


def _msg_call(srcp, dstp, hp, zrows, F, stage):
    @functools.partial(
        pl.kernel,
        out_type=jax.ShapeDtypeStruct((2, P, F), _f32),
        mesh=_mesh(),
        compiler_params=pltpu.CompilerParams(use_tc_tiling_on_sc=False),
        scratch_types=[
            pltpu.VMEM((NCH, CHUNK), _i32),
            pltpu.VMEM((NCH, CHUNK), _i32),
            pltpu.VMEM((CHUNK, F), _f32),
            pltpu.VMEM((CHUNK, F), _f32),
            pltpu.VMEM((CHUNK, F), _f32),
            pltpu.VMEM((CHUNK, F), _f32),
            pltpu.VMEM_SHARED((P, F if stage else 1), _f32),
            pltpu.VMEM_SHARED((P, F), _f32),
            pltpu.SemaphoreType.DMA,
            pltpu.SemaphoreType.DMA,
            pltpu.SemaphoreType.DMA,
            pltpu.SemaphoreType.DMA,
            pltpu.SemaphoreType.DMA,
            pltpu.SemaphoreType.DMA,
            pltpu.SemaphoreType.DMA,
            pltpu.SemaphoreType.DMA,
        ],
    )
    def msg_kernel(src_hbm, dst_hbm, h_hbm, z_hbm, out_hbm,
                   srcv, dstv, g0, g1, g2, g3, hs, acc,
                   sg0, sg1, sg2, sg3, ss0, ss1, ss2, ss3):
        c = lax.axis_index("c")
        s = lax.axis_index("s")
        wid = s * 2 + c
        gb = [g0, g1, g2, g3]
        sg = [sg0, sg1, sg2, sg3]
        ss = [ss0, ss1, ss2, ss3]
        d0 = pltpu.async_copy(z_hbm, acc.at[pl.ds(s * RPT, RPT)], sg0)
        if stage:
            d1 = pltpu.async_copy(h_hbm.at[pl.ds(s * RPT, RPT)],
                                  hs.at[pl.ds(s * RPT, RPT)], sg1)
        htab = hs if stage else h_hbm
        d2 = pltpu.async_copy(src_hbm.at[wid], srcv, sg2)
        d3 = pltpu.async_copy(dst_hbm.at[wid], dstv, sg3)
        d0.wait()
        if stage:
            d1.wait()
        d2.wait()
        d3.wait()
        plsc.subcore_barrier()

        for b in range(4):
            pltpu.async_copy(htab.at[srcv.at[b]], gb[b], sg[b])

        def body(g, carry):
            for b in range(4):
                j = g * 4 + b
                pltpu.make_async_copy(htab.at[srcv.at[j]], gb[b], sg[b]).wait()
                pltpu.async_copy(gb[b], acc.at[dstv.at[j]], ss[b], add=True)
                bp = (b + 3) & 3
                ok = jnp.logical_and(j >= 1, j <= NCH - 4)

                @pl.when(ok)
                def _():
                    pltpu.make_async_copy(gb[bp], acc.at[dstv.at[0]],
                                          ss[bp]).wait()
                    pltpu.async_copy(htab.at[srcv.at[j + 3]], gb[bp], sg[bp])

            return carry

        lax.fori_loop(0, NCH // 4, body, 0)
        for b in range(4):
            pltpu.make_async_copy(gb[b], acc.at[dstv.at[0]], ss[b]).wait()
        plsc.subcore_barrier()
        pltpu.sync_copy(acc.at[pl.ds(s * RPT, RPT)],
                        out_hbm.at[c, pl.ds(s * RPT, RPT)])

    return msg_kernel(srcp, dstp, hp, zrows)


def _msg2_call(srcp, dstp, h2a, h2b, zrows):
    F = 32

    @functools.partial(
        pl.kernel,
        out_type=jax.ShapeDtypeStruct((2, 2, P, F), _f32),
        mesh=_mesh(),
        compiler_params=pltpu.CompilerParams(use_tc_tiling_on_sc=False),
        scratch_types=[
            pltpu.VMEM((NCH, CHUNK), _i32),
            pltpu.VMEM((NCH, CHUNK), _i32),
            pltpu.VMEM((CHUNK, F), _f32),
            pltpu.VMEM((CHUNK, F), _f32),
            pltpu.VMEM((CHUNK, F), _f32),
            pltpu.VMEM((CHUNK, F), _f32),
            pltpu.VMEM_SHARED((P, F), _f32),
            pltpu.VMEM_SHARED((P, F), _f32),
            pltpu.SemaphoreType.DMA,
            pltpu.SemaphoreType.DMA,
            pltpu.SemaphoreType.DMA,
            pltpu.SemaphoreType.DMA,
            pltpu.SemaphoreType.DMA,
            pltpu.SemaphoreType.DMA,
            pltpu.SemaphoreType.DMA,
            pltpu.SemaphoreType.DMA,
        ],
    )
    def msg2_kernel(src_hbm, dst_hbm, ha_hbm, hb_hbm, z_hbm, out_hbm,
                    srcv, dstv, g0, g1, g2, g3, hs, acc,
                    sg0, sg1, sg2, sg3, ss0, ss1, ss2, ss3):
        c = lax.axis_index("c")
        s = lax.axis_index("s")
        wid = s * 2 + c
        gb = [g0, g1, g2, g3]
        sg = [sg0, sg1, sg2, sg3]
        ss = [ss0, ss1, ss2, ss3]
        halves = [ha_hbm, hb_hbm]
        pltpu.sync_copy(src_hbm.at[wid], srcv)
        pltpu.sync_copy(dst_hbm.at[wid], dstv)

        for half in range(2):
            pltpu.sync_copy(z_hbm, acc.at[pl.ds(s * RPT, RPT)])
            pltpu.sync_copy(halves[half].at[pl.ds(s * RPT, RPT)],
                            hs.at[pl.ds(s * RPT, RPT)])
            plsc.subcore_barrier()

            for b in range(4):
                pltpu.async_copy(hs.at[srcv.at[b]], gb[b], sg[b])

            def body(g, carry):
                for b in range(4):
                    j = g * 4 + b
                    pltpu.make_async_copy(hs.at[srcv.at[j]], gb[b],
                                          sg[b]).wait()
                    pltpu.async_copy(gb[b], acc.at[dstv.at[j]], ss[b],
                                     add=True)
                    bp = (b + 3) & 3
                    ok = jnp.logical_and(j >= 1, j <= NCH - 4)

                    @pl.when(ok)
                    def _():
                        pltpu.make_async_copy(gb[bp], acc.at[dstv.at[0]],
                                              ss[bp]).wait()
                        pltpu.async_copy(hs.at[srcv.at[j + 3]], gb[bp],
                                         sg[bp])

                return carry

            lax.fori_loop(0, NCH // 4, body, 0)
            for b in range(4):
                pltpu.make_async_copy(gb[b], acc.at[dstv.at[0]], ss[b]).wait()
            plsc.subcore_barrier()
            pltpu.sync_copy(acc.at[pl.ds(s * RPT, RPT)],
                            out_hbm.at[c, half, pl.ds(s * RPT, RPT)])

    return msg2_kernel(srcp, dstp, h2a, h2b, zrows)


def _tcA(deg2, x_p, W1):
    def body(deg_ref, x_ref, w_ref, dinv_ref, h_ref):
        d = deg_ref[...]
        deg = (d[0] + d[1])[:, :1]
        dinv = lax.rsqrt(deg + 1.0)
        dinv_ref[...] = dinv
        h_ref[...] = jnp.dot(x_ref[...], w_ref[...],
                             preferred_element_type=_f32) * dinv

    return pl.pallas_call(
        body,
        out_shape=[jax.ShapeDtypeStruct((P, 1), _f32),
                   jax.ShapeDtypeStruct((P, W1.shape[1]), _f32)],
    )(deg2, x_p, W1)


def _tcB(s_parts, hp, dinv, b, g, be, Wn):
    Fn = Wn.shape[1]
    npart = len(s_parts)

    def body(*refs):
        s_refs = refs[:npart]
        (hp_ref, dinv_ref, b_ref, g_ref, be_ref, w_ref, out_ref) = refs[npart:]
        dinv = dinv_ref[...]
        stot = jnp.concatenate([r[0] + r[1] for r in s_refs], axis=1)
        pre = (stot + hp_ref[...]) * dinv + b_ref[...]
        rows = lax.broadcasted_iota(_i32, (P, 1), 0)
        m = rows < N
        prem = jnp.where(m, pre, 0.0)
        mean = jnp.sum(prem, axis=0, keepdims=True) * (1.0 / N)
        d = pre - mean
        var = jnp.sum(jnp.where(m, d * d, 0.0), axis=0, keepdims=True) * (1.0 / N)
        h = jnp.maximum(g_ref[...] * d * lax.rsqrt(var + 1e-5) + be_ref[...], 0.0)
        out_ref[...] = jnp.dot(h, w_ref[...], preferred_element_type=_f32) * dinv

    return pl.pallas_call(
        body,
        out_shape=jax.ShapeDtypeStruct((P, Fn), _f32),
    )(*s_parts, hp, dinv, b, g, be, Wn)


def _tcC(s2, hp, dinv, b, g, be, batch_col, Wf1, bf1, Wf2, bf2):
    def body(s_ref, hp_ref, dinv_ref, b_ref, g_ref, be_ref, batch_ref,
             wf1_ref, bf1_ref, wf2_ref, bf2_ref, out_ref):
        dinv = dinv_ref[...]
        pre = (s_ref[0] + s_ref[1] + hp_ref[...]) * dinv + b_ref[...]
        rows = lax.broadcasted_iota(_i32, (P, 1), 0)
        m = rows < N
        prem = jnp.where(m, pre, 0.0)
        mean = jnp.sum(prem, axis=0, keepdims=True) * (1.0 / N)
        d = pre - mean
        var = jnp.sum(jnp.where(m, d * d, 0.0), axis=0, keepdims=True) * (1.0 / N)
        h = jnp.maximum(g_ref[...] * d * lax.rsqrt(var + 1e-5) + be_ref[...], 0.0)
        bcol = batch_ref[...]
        sums = []
        cnts = []
        for gi in range(G):
            sel = bcol == gi
            sums.append(jnp.sum(jnp.where(sel, h, 0.0), axis=0, keepdims=True))
            cnts.append(jnp.sum(jnp.where(sel, 1.0, 0.0), axis=0, keepdims=True))
        pooled = jnp.concatenate(sums, axis=0) / jnp.maximum(
            jnp.concatenate(cnts, axis=0), 1.0)
        o = jnp.maximum(jnp.dot(pooled, wf1_ref[...],
                                preferred_element_type=_f32) + bf1_ref[...], 0.0)
        out_ref[...] = jnp.dot(o, wf2_ref[...],
                               preferred_element_type=_f32) + bf2_ref[...]

    return pl.pallas_call(
        body,
        out_shape=jax.ShapeDtypeStruct((G, 1), _f32),
    )(s2, hp, dinv, b, g, be, batch_col, Wf1, bf1, Wf2, bf2)


def kernel(x, edge_index, batch, W1, b1, g1, be1, W2, b2, g2, be2,
           W3, b3, g3, be3, Wf1, bf1, Wf2, bf2):
    src = edge_index[0]
    dst = edge_index[1]
    pad = jnp.full((EP - E,), N, _i32)
    srcp = jnp.concatenate([src, pad]).reshape(NW, NCH, CHUNK)
    dstp = jnp.concatenate([dst, pad]).reshape(NW, NCH, CHUNK)
    x_p = jnp.zeros((P, D), _f32).at[:N].set(x)
    z16 = jnp.zeros((RPT, 8), _f32)
    z32 = jnp.zeros((RPT, 32), _f32)
    ones_rows = jnp.ones((CHUNK, 8), _f32)

    deg2 = _deg_call(dstp, ones_rows, z16)

    dinv, h1p = _tcA(deg2, x_p, W1)
    s1 = _msg_call(srcp, dstp, h1p, z32, 32, True)
    h2p = _tcB([s1], h1p, dinv, b1.reshape(1, -1), g1.reshape(1, -1),
               be1.reshape(1, -1), W2)
    s2a = _msg_call(srcp, dstp, h2p[:, :32], z32, 32, True)
    s2b = _msg_call(srcp, dstp, h2p[:, 32:], z32, 32, True)
    h3p = _tcB([s2a, s2b], h2p, dinv,
               b2.reshape(1, -1), g2.reshape(1, -1),
               be2.reshape(1, -1), W3)
    s3 = _msg_call(srcp, dstp, h3p, z32, 32, True)

    batch_col = jnp.full((P,), -1, _i32).at[:N].set(batch).reshape(P, 1)
    return _tcC(s3, h3p, dinv, b3.reshape(1, -1), g3.reshape(1, -1),
                be3.reshape(1, -1), batch_col, Wf1, bf1.reshape(1, -1),
                Wf2, bf2.reshape(1, 1))
